# Initial kernel scaffold; baseline (speedup 1.0000x reference)
import functools

import jax
import jax.numpy as jnp
import numpy as np
from jax import lax
from jax.experimental import pallas as pl
from jax.experimental.pallas import tpu as pltpu
from jax.experimental.pallas import tpu_sc as plsc

N_NODES_REAL = 10000
D_IN = 128
D_OUT = 64
N_EDGES_REAL = 320000

NC, NS = 2, 16
NW = NC * NS
CH = 128
N_PAD = 10240
JUNK = N_NODES_REAL
N_IT = (N_EDGES_REAL + NW * CH - 1) // (NW * CH)
E_PAD = NW * CH * N_IT
ROWS_PER_SC_TILE = N_PAD // NS
INIT_CHUNKS = ROWS_PER_SC_TILE // CH

_MULT_I32 = jnp.asarray(
    np.random.default_rng(0)
    .integers(1, 2**31 - 1, size=(D_IN,))
    .astype(np.int64)
    .astype(np.int32)
).reshape(1, D_IN)

_mesh = plsc.VectorSubcoreMesh(core_axis_name="c", subcore_axis_name="s")


@functools.partial(
    pl.kernel,
    mesh=_mesh,
    out_type=jax.ShapeDtypeStruct((NC * N_PAD, 16), jnp.float32),
    scratch_types=[
        pltpu.VMEM((CH,), jnp.int32),
        pltpu.VMEM((CH, 16), jnp.float32),
        pltpu.VMEM((CH, 16), jnp.float32),
        pltpu.VMEM_SHARED((N_PAD, 16), jnp.float32),
        pltpu.SemaphoreType.DMA,
    ],
)
def _sc_degree(dst_hbm, ones_hbm, zeros_hbm, out_hbm,
               idx_v, ones_v, zeros_v, acc, sem):
    c = lax.axis_index("c")
    s = lax.axis_index("s")
    pltpu.sync_copy(ones_hbm, ones_v)
    pltpu.sync_copy(zeros_hbm, zeros_v)
    for k in range(INIT_CHUNKS):
        row0 = s * ROWS_PER_SC_TILE + k * CH

        @pl.when(c == 0)
        def _():
            pltpu.sync_copy(ones_v, acc.at[pl.ds(row0, CH)])

        @pl.when(c != 0)
        def _():
            pltpu.sync_copy(zeros_v, acc.at[pl.ds(row0, CH)])

    plsc.subcore_barrier()
    wid = s * NC + c

    def body(it, carry):
        base = wid * (CH * N_IT) + it * CH
        pltpu.sync_copy(dst_hbm.at[pl.ds(base, CH)], idx_v)
        pltpu.sync_copy(ones_v, acc.at[idx_v], add=True)
        return carry

    lax.fori_loop(0, N_IT, body, 0)
    plsc.subcore_barrier()
    for k in range(INIT_CHUNKS):
        row0 = s * ROWS_PER_SC_TILE + k * CH
        pltpu.sync_copy(acc.at[pl.ds(row0, CH)],
                        out_hbm.at[pl.ds(c * N_PAD + row0, CH)])


@functools.partial(
    pl.kernel,
    mesh=_mesh,
    out_type=jax.ShapeDtypeStruct((NC * N_PAD, D_IN), jnp.float32),
    scratch_types=[
        pltpu.VMEM((CH,), jnp.int32),
        pltpu.VMEM((CH,), jnp.int32),
        pltpu.VMEM((CH, D_IN), jnp.float32),
        pltpu.VMEM((CH, D_IN), jnp.float32),
        pltpu.VMEM_SHARED((N_PAD, D_IN), jnp.float32),
        pltpu.SemaphoreType.DMA,
    ],
)
def _sc_hop(y_hbm, src_hbm, dst_hbm, zeros_hbm, out_hbm,
            src_v, dst_v, rows_v, zrows_v, acc, sem):
    c = lax.axis_index("c")
    s = lax.axis_index("s")
    pltpu.sync_copy(zeros_hbm, zrows_v)
    for k in range(INIT_CHUNKS):
        row0 = s * ROWS_PER_SC_TILE + k * CH

        @pl.when(c == 0)
        def _():
            pltpu.sync_copy(y_hbm.at[pl.ds(row0, CH)], rows_v)
            pltpu.sync_copy(rows_v, acc.at[pl.ds(row0, CH)])

        @pl.when(c != 0)
        def _():
            pltpu.sync_copy(zrows_v, acc.at[pl.ds(row0, CH)])

    plsc.subcore_barrier()
    wid = s * NC + c

    def body(it, carry):
        base = wid * (CH * N_IT) + it * CH
        pltpu.sync_copy(src_hbm.at[pl.ds(base, CH)], src_v)
        pltpu.sync_copy(dst_hbm.at[pl.ds(base, CH)], dst_v)
        pltpu.async_copy(y_hbm.at[src_v], rows_v, sem).wait()
        pltpu.sync_copy(rows_v, acc.at[dst_v], add=True)
        return carry

    lax.fori_loop(0, N_IT, body, 0)
    plsc.subcore_barrier()
    for k in range(INIT_CHUNKS):
        row0 = s * ROWS_PER_SC_TILE + k * CH
        pltpu.sync_copy(acc.at[pl.ds(row0, CH)],
                        out_hbm.at[pl.ds(c * N_PAD + row0, CH)])


GCH = 64


@functools.partial(
    pl.kernel,
    mesh=_mesh,
    out_type=jax.ShapeDtypeStruct((N_PAD, D_OUT), jnp.float32),
    scratch_types=[
        pltpu.VMEM((GCH,), jnp.int32),
        pltpu.VMEM((GCH, D_OUT), jnp.float32),
        pltpu.SemaphoreType.DMA,
    ],
)
def _sc_gather_rows(ls_hbm, rep_hbm, out_hbm, idx_v, rows_v, sem):
    c = lax.axis_index("c")
    s = lax.axis_index("s")
    wid = s * NC + c
    for k in range(N_PAD // NW // GCH):
        base = wid * (N_PAD // NW) + k * GCH
        pltpu.sync_copy(rep_hbm.at[pl.ds(base, GCH)], idx_v)
        pltpu.async_copy(ls_hbm.at[idx_v], rows_v, sem).wait()
        pltpu.sync_copy(rows_v, out_hbm.at[pl.ds(base, GCH)])


def _tc_prep(degp, x_pad):
    def body(degp_ref, x_ref, dinv_ref, y0_ref):
        d = degp_ref[0, :, 0:1] + degp_ref[1, :, 0:1]
        dinv = lax.rsqrt(jnp.maximum(d, 1.0))
        dinv_ref[...] = dinv
        y0_ref[...] = x_ref[...] * dinv

    return pl.pallas_call(
        body,
        out_shape=(
            jax.ShapeDtypeStruct((N_PAD, 1), jnp.float32),
            jax.ShapeDtypeStruct((N_PAD, D_IN), jnp.float32),
        ),
    )(degp, x_pad)


def _tc_scale2(sp, dinv):
    def body(sp_ref, dinv_ref, y1_ref):
        dv = dinv_ref[...]
        y1_ref[...] = (sp_ref[0] + sp_ref[1]) * (dv * dv)

    return pl.pallas_call(
        body,
        out_shape=jax.ShapeDtypeStruct((N_PAD, D_IN), jnp.float32),
    )(sp, dinv)


def _tc_mlp(sp2, dinv, W1, b1, W2, b2, mult):
    def body(sp_ref, dinv_ref, w1_ref, b1_ref, w2_ref, b2_ref, m_ref,
             ls_ref, h_ref):
        xp = (sp_ref[0] + sp_ref[1]) * dinv_ref[...]
        keys = jnp.round(xp).astype(jnp.int32)
        h_ref[...] = jnp.sum(keys * m_ref[...], axis=1, keepdims=True,
                             dtype=jnp.int32)
        hid = jnp.maximum(
            lax.dot_general(xp, w1_ref[...], (((1,), (1,)), ((), ())),
                            preferred_element_type=jnp.float32)
            + b1_ref[...], 0.0)
        o = lax.dot_general(hid, w2_ref[...], (((1,), (1,)), ((), ())),
                            preferred_element_type=jnp.float32) + b2_ref[...]
        o = o - jnp.max(o, axis=1, keepdims=True)
        ls_ref[...] = o - jnp.log(jnp.sum(jnp.exp(o), axis=1, keepdims=True))

    return pl.pallas_call(
        body,
        out_shape=(
            jax.ShapeDtypeStruct((N_PAD, D_OUT), jnp.float32),
            jax.ShapeDtypeStruct((N_PAD, 1), jnp.int32),
        ),
    )(sp2, dinv, W1, b1.reshape(1, D_IN), W2, b2.reshape(1, D_OUT), mult)


REP_BI = 1024
REP_BJ = 512


def _tc_rep(h_col, h_row):
    BIG = jnp.int32(2**30)

    def body(hA_ref, hB_ref, rep_ref):
        hi = hA_ref[...]
        best = jnp.full((REP_BI, 1), BIG, jnp.int32)
        for k in range(N_PAD // REP_BJ):
            hj = hB_ref[:, k * REP_BJ:(k + 1) * REP_BJ]
            eq = hi == hj
            jidx = lax.broadcasted_iota(jnp.int32, (REP_BI, REP_BJ), 1) \
                + jnp.int32(k * REP_BJ)
            cand = jnp.where(eq, jidx, BIG)
            best = jnp.minimum(best, jnp.min(cand, axis=1, keepdims=True))
        rep_ref[...] = best

    return pl.pallas_call(
        body,
        grid=(N_PAD // REP_BI,),
        in_specs=[
            pl.BlockSpec((REP_BI, 1), lambda i: (i, 0)),
            pl.BlockSpec((1, N_PAD), lambda i: (0, 0)),
        ],
        out_specs=pl.BlockSpec((REP_BI, 1), lambda i: (i, 0)),
        out_shape=jax.ShapeDtypeStruct((N_PAD, 1), jnp.int32),
    )(h_col, h_row)


def kernel(x, edge_index, W1, b1, W2, b2):
    x = x.astype(jnp.float32)
    ei = edge_index.astype(jnp.int32)
    n_extra = E_PAD - N_EDGES_REAL
    src = jnp.concatenate([ei[0], jnp.zeros((n_extra,), jnp.int32)])
    dst = jnp.concatenate([ei[1], jnp.full((n_extra,), JUNK, jnp.int32)])
    x_pad = jnp.pad(x, ((0, N_PAD - N_NODES_REAL), (0, 0)))
    ones16 = jnp.ones((CH, 16), jnp.float32)
    zeros16 = jnp.zeros((CH, 16), jnp.float32)
    zeros128 = jnp.zeros((CH, D_IN), jnp.float32)

    degp = _sc_degree(dst, ones16, zeros16).reshape(2, N_PAD, 16)
    dinv, y0 = _tc_prep(degp, x_pad)
    sp1 = _sc_hop(y0, src, dst, zeros128).reshape(2, N_PAD, D_IN)
    y1 = _tc_scale2(sp1, dinv)
    sp2 = _sc_hop(y1, src, dst, zeros128).reshape(2, N_PAD, D_IN)
    ls, h = _tc_mlp(sp2, dinv, W1, b1, W2, b2, _MULT_I32)
    rep = _tc_rep(h, h.reshape(1, N_PAD))
    out = _sc_gather_rows(ls, rep.reshape(N_PAD))
    return out[:N_NODES_REAL]


# speedup vs baseline: 7.2169x; 7.2169x over previous
import functools

import jax
import jax.numpy as jnp
import numpy as np
from jax import lax
from jax.experimental import pallas as pl
from jax.experimental.pallas import tpu as pltpu
from jax.experimental.pallas import tpu_sc as plsc

N_NODES_REAL = 10000
D_IN = 128
D_OUT = 64
N_EDGES_REAL = 320000

NC, NS = 2, 16
NW = NC * NS
CH = 128
N_PAD = 10240
JUNK = N_NODES_REAL
N_IT = (N_EDGES_REAL + NW * CH - 1) // (NW * CH)
E_PAD = NW * CH * N_IT
ROWS_PER_SC_TILE = N_PAD // NS
INIT_CHUNKS = ROWS_PER_SC_TILE // CH

_MULT_I32 = (
    np.random.default_rng(0)
    .integers(1, 2**31 - 1, size=(D_IN,))
    .astype(np.int64)
    .astype(np.int32)
    .reshape(1, D_IN)
)

_mesh = plsc.VectorSubcoreMesh(core_axis_name="c", subcore_axis_name="s")


@functools.partial(
    pl.kernel,
    mesh=_mesh,
    out_type=jax.ShapeDtypeStruct((NC * N_PAD, D_IN), jnp.float32),
    scratch_types=[
        pltpu.VMEM((CH,), jnp.int32),
        pltpu.VMEM((CH, D_IN), jnp.float32),
        pltpu.VMEM((CH, D_IN), jnp.float32),
        pltpu.VMEM_SHARED((N_PAD, D_IN), jnp.float32),
        pltpu.SemaphoreType.DMA,
    ],
)
def _sc_degree(dst_hbm, ones_hbm, zeros_hbm, out_hbm,
               idx_v, ones_v, zeros_v, acc, sem):
    c = lax.axis_index("c")
    s = lax.axis_index("s")
    pltpu.sync_copy(ones_hbm, ones_v)
    pltpu.sync_copy(zeros_hbm, zeros_v)
    for k in range(INIT_CHUNKS):
        row0 = s * ROWS_PER_SC_TILE + k * CH

        @pl.when(c == 0)
        def _():
            pltpu.sync_copy(ones_v, acc.at[pl.ds(row0, CH)])

        @pl.when(c != 0)
        def _():
            pltpu.sync_copy(zeros_v, acc.at[pl.ds(row0, CH)])

    plsc.subcore_barrier()
    wid = s * NC + c

    def body(it, carry):
        base = wid * (CH * N_IT) + it * CH
        pltpu.sync_copy(dst_hbm.at[pl.ds(base, CH)], idx_v)
        pltpu.sync_copy(ones_v, acc.at[idx_v], add=True)
        return carry

    lax.fori_loop(0, N_IT, body, 0)
    plsc.subcore_barrier()
    for k in range(INIT_CHUNKS):
        row0 = s * ROWS_PER_SC_TILE + k * CH
        pltpu.sync_copy(acc.at[pl.ds(row0, CH)],
                        out_hbm.at[pl.ds(c * N_PAD + row0, CH)])


@functools.partial(
    pl.kernel,
    mesh=_mesh,
    out_type=jax.ShapeDtypeStruct((NC * N_PAD, D_IN), jnp.float32),
    scratch_types=[
        pltpu.VMEM((CH,), jnp.int32),
        pltpu.VMEM((CH,), jnp.int32),
        pltpu.VMEM((CH, D_IN), jnp.float32),
        pltpu.VMEM((CH, D_IN), jnp.float32),
        pltpu.VMEM_SHARED((N_PAD, D_IN), jnp.float32),
        pltpu.SemaphoreType.DMA,
    ],
)
def _sc_hop(y_hbm, src_hbm, dst_hbm, zeros_hbm, out_hbm,
            src_v, dst_v, rows_v, zrows_v, acc, sem):
    c = lax.axis_index("c")
    s = lax.axis_index("s")
    pltpu.sync_copy(zeros_hbm, zrows_v)
    for k in range(INIT_CHUNKS):
        row0 = s * ROWS_PER_SC_TILE + k * CH

        @pl.when(c == 0)
        def _():
            pltpu.sync_copy(y_hbm.at[pl.ds(row0, CH)], rows_v)
            pltpu.sync_copy(rows_v, acc.at[pl.ds(row0, CH)])

        @pl.when(c != 0)
        def _():
            pltpu.sync_copy(zrows_v, acc.at[pl.ds(row0, CH)])

    plsc.subcore_barrier()
    wid = s * NC + c

    def body(it, carry):
        base = wid * (CH * N_IT) + it * CH
        pltpu.sync_copy(src_hbm.at[pl.ds(base, CH)], src_v)
        pltpu.sync_copy(dst_hbm.at[pl.ds(base, CH)], dst_v)
        pltpu.async_copy(y_hbm.at[src_v], rows_v, sem).wait()
        pltpu.sync_copy(rows_v, acc.at[dst_v], add=True)
        return carry

    lax.fori_loop(0, N_IT, body, 0)
    plsc.subcore_barrier()
    for k in range(INIT_CHUNKS):
        row0 = s * ROWS_PER_SC_TILE + k * CH
        pltpu.sync_copy(acc.at[pl.ds(row0, CH)],
                        out_hbm.at[pl.ds(c * N_PAD + row0, CH)])


GCH = 64


@functools.partial(
    pl.kernel,
    mesh=_mesh,
    out_type=jax.ShapeDtypeStruct((N_PAD, D_IN), jnp.float32),
    scratch_types=[
        pltpu.VMEM((GCH,), jnp.int32),
        pltpu.VMEM((GCH, D_IN), jnp.float32),
        pltpu.SemaphoreType.DMA,
    ],
)
def _sc_gather_rows(ls_hbm, rep_hbm, out_hbm, idx_v, rows_v, sem):
    c = lax.axis_index("c")
    s = lax.axis_index("s")
    wid = s * NC + c
    for k in range(N_PAD // NW // GCH):
        base = wid * (N_PAD // NW) + k * GCH
        pltpu.sync_copy(rep_hbm.at[pl.ds(base, GCH)], idx_v)
        pltpu.async_copy(ls_hbm.at[idx_v], rows_v, sem).wait()
        pltpu.sync_copy(rows_v, out_hbm.at[pl.ds(base, GCH)])


def _tc_prep(degp, x_pad):
    def body(degp_ref, x_ref, dinv_ref, y0_ref):
        d = degp_ref[0, :, 0:1] + degp_ref[1, :, 0:1]
        dinv = lax.rsqrt(jnp.maximum(d, 1.0))
        dinv_ref[...] = dinv
        y0_ref[...] = x_ref[...] * dinv

    return pl.pallas_call(
        body,
        out_shape=(
            jax.ShapeDtypeStruct((N_PAD, 1), jnp.float32),
            jax.ShapeDtypeStruct((N_PAD, D_IN), jnp.float32),
        ),
    )(degp, x_pad)


def _tc_scale2(sp, dinv):
    def body(sp_ref, dinv_ref, y1_ref):
        dv = dinv_ref[...]
        y1_ref[...] = (sp_ref[0] + sp_ref[1]) * (dv * dv)

    return pl.pallas_call(
        body,
        out_shape=jax.ShapeDtypeStruct((N_PAD, D_IN), jnp.float32),
    )(sp, dinv)


def _tc_mlp(sp2, dinv, W1, b1, W2, b2, mult):
    def body(sp_ref, dinv_ref, w1_ref, b1_ref, w2_ref, b2_ref, m_ref,
             ls_ref, h_ref):
        xp = (sp_ref[0] + sp_ref[1]) * dinv_ref[...]
        keys = jnp.round(xp).astype(jnp.int32)
        h_ref[...] = jnp.sum(keys * m_ref[...], axis=1, keepdims=True,
                             dtype=jnp.int32)
        hid = jnp.maximum(
            lax.dot_general(xp, w1_ref[...], (((1,), (1,)), ((), ())),
                            preferred_element_type=jnp.float32)
            + b1_ref[...], 0.0)
        o = lax.dot_general(hid, w2_ref[...], (((1,), (1,)), ((), ())),
                            preferred_element_type=jnp.float32) + b2_ref[...]
        o = o - jnp.max(o, axis=1, keepdims=True)
        ls = o - jnp.log(jnp.sum(jnp.exp(o), axis=1, keepdims=True))
        ls_ref[...] = jnp.concatenate([ls, jnp.zeros_like(ls)], axis=1)

    return pl.pallas_call(
        body,
        out_shape=(
            jax.ShapeDtypeStruct((N_PAD, D_IN), jnp.float32),
            jax.ShapeDtypeStruct((N_PAD, 1), jnp.int32),
        ),
    )(sp2, dinv, W1, b1.reshape(1, D_IN), W2, b2.reshape(1, D_OUT), mult)


REP_BI = 1024
REP_BJ = 512


def _tc_rep(h_col, h_row):
    BIG = 2**30

    def body(hA_ref, hB_ref, rep_ref):
        hi = hA_ref[...]
        best = jnp.full((REP_BI, 1), BIG, jnp.int32)
        for k in range(N_PAD // REP_BJ):
            hj = hB_ref[:, k * REP_BJ:(k + 1) * REP_BJ]
            eq = hi == hj
            jidx = lax.broadcasted_iota(jnp.int32, (REP_BI, REP_BJ), 1) \
                + jnp.int32(k * REP_BJ)
            cand = jnp.where(eq, jidx, BIG)
            best = jnp.minimum(best, jnp.min(cand, axis=1, keepdims=True))
        rep_ref[...] = best

    return pl.pallas_call(
        body,
        grid=(N_PAD // REP_BI,),
        in_specs=[
            pl.BlockSpec((REP_BI, 1), lambda i: (i, 0)),
            pl.BlockSpec((1, N_PAD), lambda i: (0, 0)),
        ],
        out_specs=pl.BlockSpec((REP_BI, 1), lambda i: (i, 0)),
        out_shape=jax.ShapeDtypeStruct((N_PAD, 1), jnp.int32),
    )(h_col, h_row)


def kernel(x, edge_index, W1, b1, W2, b2):
    x = x.astype(jnp.float32)
    ei = edge_index.astype(jnp.int32)
    n_extra = E_PAD - N_EDGES_REAL
    src = jnp.concatenate([ei[0], jnp.zeros((n_extra,), jnp.int32)])
    dst = jnp.concatenate([ei[1], jnp.full((n_extra,), JUNK, jnp.int32)])
    x_pad = jnp.pad(x, ((0, N_PAD - N_NODES_REAL), (0, 0)))
    ones128 = jnp.ones((CH, D_IN), jnp.float32)
    zeros128 = jnp.zeros((CH, D_IN), jnp.float32)

    degp = _sc_degree(dst, ones128, zeros128).reshape(2, N_PAD, D_IN)
    dinv, y0 = _tc_prep(degp, x_pad)
    sp1 = _sc_hop(y0, src, dst, zeros128).reshape(2, N_PAD, D_IN)
    y1 = _tc_scale2(sp1, dinv)
    sp2 = _sc_hop(y1, src, dst, zeros128).reshape(2, N_PAD, D_IN)
    ls, h = _tc_mlp(sp2, dinv, W1, b1, W2, b2,
                    jnp.asarray(_MULT_I32, dtype=jnp.int32))
    rep = _tc_rep(h, h.reshape(1, N_PAD))
    out = _sc_gather_rows(ls, rep.reshape(N_PAD))
    return out[:N_NODES_REAL, :D_OUT]
